# baseline (speedup 1.0000x reference)
import functools

import jax
import jax.numpy as jnp
from jax import lax
from jax.experimental import pallas as pl
from jax.experimental.pallas import tpu as pltpu
from jax.experimental.pallas import tpu_sc as plsc

N = 10000
D = 128
NC = 2
NS = 16
NW = NC * NS
CHUNK = 128
ROWS_PAD = 10240
RPT = ROWS_PAD // NS
ROWS_AGG = 10112
RPT_AGG = ROWS_AGG // NS


def _sc_mesh():
    return plsc.VectorSubcoreMesh(core_axis_name="c", subcore_axis_name="s")


_DEG_WIN = 4


def _make_deg_kernel(e_pad):
    ept = e_pad // NW
    n_chunks = ept // CHUNK

    @functools.partial(
        pl.kernel,
        out_type=jax.ShapeDtypeStruct((NC, ROWS_PAD), jnp.float32),
        mesh=_sc_mesh(),
        scratch_types=[
            pltpu.VMEM((n_chunks, CHUNK), jnp.int32),
            pltpu.VMEM((CHUNK,), jnp.float32),
            pltpu.VMEM_SHARED((ROWS_PAD,), jnp.float32),
            pltpu.SemaphoreType.DMA,
        ],
    )
    def deg_kernel(dst_hbm, zeros_hbm, out_hbm, didx_all, ones_v, acc_sh, sem):
        c = lax.axis_index("c")
        s = lax.axis_index("s")
        wid = s * NC + c
        pltpu.sync_copy(zeros_hbm.at[pl.ds(s * RPT, RPT)],
                        acc_sh.at[pl.ds(s * RPT, RPT)])
        for i in range(CHUNK // 16):
            ones_v[pl.ds(i * 16, 16)] = jnp.ones((16,), jnp.float32)
        pltpu.sync_copy(dst_hbm.at[wid], didx_all)
        plsc.subcore_barrier()

        def body(j, carry):
            pltpu.async_copy(ones_v, acc_sh.at[didx_all.at[j]], sem, add=True)

            @pl.when(j >= _DEG_WIN)
            def _():
                pltpu.make_async_copy(
                    ones_v, acc_sh.at[didx_all.at[0]], sem).wait()

            return carry

        lax.fori_loop(0, n_chunks, body, 0)
        for _ in range(min(_DEG_WIN, n_chunks)):
            pltpu.make_async_copy(ones_v, acc_sh.at[didx_all.at[0]], sem).wait()
        plsc.subcore_barrier()
        pltpu.sync_copy(acc_sh.at[pl.ds(s * RPT, RPT)],
                        out_hbm.at[c, pl.ds(s * RPT, RPT)])

    return deg_kernel


def _make_agg_kernel(e_pad):
    ept = e_pad // NW
    n_chunks = ept // CHUNK
    assert n_chunks % 3 == 0

    @functools.partial(
        pl.kernel,
        out_type=jax.ShapeDtypeStruct((NC, ROWS_AGG, D), jnp.float32),
        mesh=_sc_mesh(),
        scratch_types=[
            pltpu.VMEM((CHUNK,), jnp.int32),
            pltpu.VMEM((CHUNK,), jnp.int32),
            pltpu.VMEM((CHUNK,), jnp.int32),
            pltpu.VMEM((CHUNK,), jnp.int32),
            pltpu.VMEM((CHUNK,), jnp.int32),
            pltpu.VMEM((CHUNK,), jnp.int32),
            pltpu.VMEM((CHUNK, D), jnp.float32),
            pltpu.VMEM((CHUNK, D), jnp.float32),
            pltpu.VMEM((CHUNK, D), jnp.float32),
            pltpu.VMEM_SHARED((ROWS_AGG, D), jnp.float32),
            pltpu.SemaphoreType.DMA,
            pltpu.SemaphoreType.DMA,
            pltpu.SemaphoreType.DMA,
            pltpu.SemaphoreType.DMA,
            pltpu.SemaphoreType.DMA,
            pltpu.SemaphoreType.DMA,
            pltpu.SemaphoreType.DMA,
            pltpu.SemaphoreType.DMA,
            pltpu.SemaphoreType.DMA,
        ],
    )
    def agg_kernel(hs_hbm, src_hbm, dst_hbm, zeros_hbm, out_hbm,
                   sidx0, sidx1, sidx2, didx0, didx1, didx2,
                   rows0, rows1, rows2, acc_sh,
                   gsem0, gsem1, gsem2, isem0, isem1, isem2,
                   dsem0, dsem1, dsem2):
        c = lax.axis_index("c")
        s = lax.axis_index("s")
        wid = s * NC + c
        base = wid * ept
        sidx = (sidx0, sidx1, sidx2)
        didx = (didx0, didx1, didx2)
        rows = (rows0, rows1, rows2)
        gsem = (gsem0, gsem1, gsem2)
        isem = (isem0, isem1, isem2)
        dsem = (dsem0, dsem1, dsem2)
        pltpu.sync_copy(zeros_hbm.at[pl.ds(s * RPT_AGG, RPT_AGG)],
                        acc_sh.at[pl.ds(s * RPT_AGG, RPT_AGG)])
        plsc.subcore_barrier()

        for k in range(3):
            pltpu.async_copy(src_hbm.at[pl.ds(base + k * CHUNK, CHUNK)],
                             sidx[k], isem[k])
            pltpu.async_copy(dst_hbm.at[pl.ds(base + k * CHUNK, CHUNK)],
                             didx[k], dsem[k])
        for k in range(3):
            pltpu.make_async_copy(src_hbm.at[pl.ds(base, CHUNK)],
                                  sidx[k], isem[k]).wait()
            pltpu.async_copy(hs_hbm.at[sidx[k]], rows[k], gsem[k])

        def chunk_step(j, k):
            pltpu.make_async_copy(hs_hbm.at[sidx[k]], rows[k],
                                  gsem[k]).wait()

            @pl.when(j + 3 < n_chunks)
            def _():
                pltpu.async_copy(
                    src_hbm.at[pl.ds(base + (j + 3) * CHUNK, CHUNK)],
                    sidx[k], isem[k])

            pltpu.make_async_copy(dst_hbm.at[pl.ds(base, CHUNK)],
                                  didx[k], dsem[k]).wait()
            pltpu.sync_copy(rows[k], acc_sh.at[didx[k]], add=True)

            @pl.when(j + 3 < n_chunks)
            def _():
                pltpu.async_copy(
                    dst_hbm.at[pl.ds(base + (j + 3) * CHUNK, CHUNK)],
                    didx[k], dsem[k])
                pltpu.make_async_copy(src_hbm.at[pl.ds(base, CHUNK)],
                                      sidx[k], isem[k]).wait()
                pltpu.async_copy(hs_hbm.at[sidx[k]], rows[k], gsem[k])

        def triple(t, carry):
            j0 = 3 * t
            chunk_step(j0, 0)
            chunk_step(j0 + 1, 1)
            chunk_step(j0 + 2, 2)
            return carry

        lax.fori_loop(0, n_chunks // 3, triple, 0)
        plsc.subcore_barrier()
        pltpu.sync_copy(acc_sh.at[pl.ds(s * RPT_AGG, RPT_AGG)],
                        out_hbm.at[c, pl.ds(s * RPT_AGG, RPT_AGG)])

    return agg_kernel


_BLK = 2000


def _k1_body(x_ref, w_ref, dega_ref, degb_ref, hs_ref, dis_ref):
    deg = dega_ref[...] + degb_ref[...] + 1.0
    dis = lax.rsqrt(deg)
    h = jnp.dot(x_ref[...], w_ref[...], preferred_element_type=jnp.float32)
    hs_ref[...] = h * dis
    dis_ref[...] = dis


def _tc_k1(x, w1, dega, degb):
    grid = (N // _BLK,)
    return pl.pallas_call(
        _k1_body,
        grid=grid,
        in_specs=[
            pl.BlockSpec((_BLK, D), lambda i: (i, 0)),
            pl.BlockSpec((D, D), lambda i: (0, 0)),
            pl.BlockSpec((_BLK, 1), lambda i: (i, 0)),
            pl.BlockSpec((_BLK, 1), lambda i: (i, 0)),
        ],
        out_specs=[
            pl.BlockSpec((_BLK, D), lambda i: (i, 0)),
            pl.BlockSpec((_BLK, 1), lambda i: (i, 0)),
        ],
        out_shape=[
            jax.ShapeDtypeStruct((N, D), jnp.float32),
            jax.ShapeDtypeStruct((N, 1), jnp.float32),
        ],
    )(x, w1, dega, degb)


def _k2_body(a0_ref, a1_ref, hs_ref, dis_ref, b_ref, w_ref, out_ref):
    dis = dis_ref[...]
    t = dis * (a0_ref[...] + a1_ref[...] + hs_ref[...]) + b_ref[...]
    t = jnp.maximum(t, 0.0)
    h2 = jnp.dot(t, w_ref[...], preferred_element_type=jnp.float32)
    out_ref[...] = h2 * dis


def _tc_k2(a0, a1, hs, dis, b1, w2):
    grid = (N // _BLK,)
    return pl.pallas_call(
        _k2_body,
        grid=grid,
        in_specs=[
            pl.BlockSpec((_BLK, D), lambda i: (i, 0)),
            pl.BlockSpec((_BLK, D), lambda i: (i, 0)),
            pl.BlockSpec((_BLK, D), lambda i: (i, 0)),
            pl.BlockSpec((_BLK, 1), lambda i: (i, 0)),
            pl.BlockSpec((1, D), lambda i: (0, 0)),
            pl.BlockSpec((D, D), lambda i: (0, 0)),
        ],
        out_specs=pl.BlockSpec((_BLK, D), lambda i: (i, 0)),
        out_shape=jax.ShapeDtypeStruct((N, D), jnp.float32),
    )(a0, a1, hs, dis, b1, w2)


def _k3_body(a0_ref, a1_ref, hs_ref, dis_ref, b_ref, out_ref):
    out_ref[...] = (dis_ref[...] * (a0_ref[...] + a1_ref[...] + hs_ref[...])
                    + b_ref[...])


def _tc_k3(a0, a1, hs, dis, b2):
    grid = (N // _BLK,)
    return pl.pallas_call(
        _k3_body,
        grid=grid,
        in_specs=[
            pl.BlockSpec((_BLK, D), lambda i: (i, 0)),
            pl.BlockSpec((_BLK, D), lambda i: (i, 0)),
            pl.BlockSpec((_BLK, D), lambda i: (i, 0)),
            pl.BlockSpec((_BLK, 1), lambda i: (i, 0)),
            pl.BlockSpec((1, D), lambda i: (0, 0)),
        ],
        out_specs=pl.BlockSpec((_BLK, D), lambda i: (i, 0)),
        out_shape=jax.ShapeDtypeStruct((N, D), jnp.float32),
    )(a0, a1, hs, dis, b2)


@jax.jit
def kernel(x, edge_index, W1, b1, W2, b2):
    e = edge_index.shape[1]
    per_tile_chunks = -(-e // (NW * CHUNK))
    per_tile_chunks = -(-per_tile_chunks // 3) * 3
    e_pad = per_tile_chunks * NW * CHUNK
    pad = e_pad - e
    n_chunks = e_pad // (NW * CHUNK)
    src = jnp.concatenate([edge_index[0], jnp.zeros((pad,), jnp.int32)])
    dst = jnp.concatenate([edge_index[1], jnp.full((pad,), N, jnp.int32)])
    dst2d = dst.reshape(NW, n_chunks, CHUNK)

    zeros1 = jnp.zeros((ROWS_PAD,), jnp.float32)
    zeros2 = jnp.zeros((ROWS_AGG, D), jnp.float32)

    deg_p = _make_deg_kernel(e_pad)(dst2d, zeros1)
    dega = deg_p[0, :N].reshape(N, 1)
    degb = deg_p[1, :N].reshape(N, 1)

    hs1, dis = _tc_k1(x, W1, dega, degb)

    agg_fn = _make_agg_kernel(e_pad)
    agg1 = agg_fn(hs1, src, dst, zeros2)
    hs2 = _tc_k2(agg1[0, :N], agg1[1, :N], hs1, dis,
                 b1.reshape(1, D), W2)

    agg2 = agg_fn(hs2, src, dst, zeros2)
    out = _tc_k3(agg2[0, :N], agg2[1, :N], hs2, dis, b2.reshape(1, D))
    return out


# speedup vs baseline: 2.0576x; 2.0576x over previous
import functools

import jax
import jax.numpy as jnp
from jax import lax
from jax.experimental import pallas as pl
from jax.experimental.pallas import tpu as pltpu
from jax.experimental.pallas import tpu_sc as plsc

N = 10000
D = 128
NC = 2
NS = 16
NW = NC * NS
CHUNK = 128
ROWS_PAD = 10240
RPT = ROWS_PAD // NS
ROWS_AGG = 10112
RPT_AGG = ROWS_AGG // NS


def _sc_mesh():
    return plsc.VectorSubcoreMesh(core_axis_name="c", subcore_axis_name="s")


_DEG_WIN = 4


def _make_deg_kernel(e_pad):
    ept = e_pad // NW
    n_chunks = ept // CHUNK

    @functools.partial(
        pl.kernel,
        out_type=jax.ShapeDtypeStruct((NC, ROWS_PAD), jnp.float32),
        mesh=_sc_mesh(),
        scratch_types=[
            pltpu.VMEM((n_chunks, CHUNK), jnp.int32),
            pltpu.VMEM((CHUNK,), jnp.float32),
            pltpu.VMEM_SHARED((ROWS_PAD,), jnp.float32),
            pltpu.SemaphoreType.DMA,
        ],
    )
    def deg_kernel(dst_hbm, zeros_hbm, out_hbm, didx_all, ones_v, acc_sh, sem):
        c = lax.axis_index("c")
        s = lax.axis_index("s")
        wid = s * NC + c
        pltpu.sync_copy(zeros_hbm.at[pl.ds(s * RPT, RPT)],
                        acc_sh.at[pl.ds(s * RPT, RPT)])
        for i in range(CHUNK // 16):
            ones_v[pl.ds(i * 16, 16)] = jnp.ones((16,), jnp.float32)
        pltpu.sync_copy(dst_hbm.at[wid], didx_all)
        plsc.subcore_barrier()

        def body(j, carry):
            pltpu.async_copy(ones_v, acc_sh.at[didx_all.at[j]], sem, add=True)

            @pl.when(j >= _DEG_WIN)
            def _():
                pltpu.make_async_copy(
                    ones_v, acc_sh.at[didx_all.at[0]], sem).wait()

            return carry

        lax.fori_loop(0, n_chunks, body, 0)
        for _ in range(min(_DEG_WIN, n_chunks)):
            pltpu.make_async_copy(ones_v, acc_sh.at[didx_all.at[0]], sem).wait()
        plsc.subcore_barrier()
        pltpu.sync_copy(acc_sh.at[pl.ds(s * RPT, RPT)],
                        out_hbm.at[c, pl.ds(s * RPT, RPT)])

    return deg_kernel


def _make_agg_kernel(e_pad):
    ept = e_pad // NW
    n_chunks = ept // CHUNK

    @functools.partial(
        pl.kernel,
        out_type=jax.ShapeDtypeStruct((NC, ROWS_AGG, D), jnp.float32),
        mesh=_sc_mesh(),
        scratch_types=[
            pltpu.VMEM((n_chunks, CHUNK), jnp.int32),
            pltpu.VMEM((CHUNK,), jnp.int32),
            pltpu.VMEM((CHUNK,), jnp.int32),
            pltpu.VMEM((CHUNK, D), jnp.float32),
            pltpu.VMEM((CHUNK, D), jnp.float32),
            pltpu.VMEM_SHARED((ROWS_AGG, D), jnp.float32),
            pltpu.SemaphoreType.DMA,
            pltpu.SemaphoreType.DMA,
            pltpu.SemaphoreType.DMA,
            pltpu.SemaphoreType.DMA,
        ],
    )
    def agg_kernel(hs_hbm, src_hbm, dst_hbm, zeros_hbm, out_hbm,
                   didx_all, sidx0, sidx1, rows0, rows1, acc_sh,
                   gsem0, gsem1, isem0, isem1):
        c = lax.axis_index("c")
        s = lax.axis_index("s")
        wid = s * NC + c
        base = wid * ept
        pltpu.sync_copy(zeros_hbm.at[pl.ds(s * RPT_AGG, RPT_AGG)],
                        acc_sh.at[pl.ds(s * RPT_AGG, RPT_AGG)])
        pltpu.sync_copy(dst_hbm.at[wid], didx_all)
        plsc.subcore_barrier()

        pltpu.async_copy(src_hbm.at[pl.ds(base, CHUNK)], sidx0, isem0)
        pltpu.async_copy(src_hbm.at[pl.ds(base + CHUNK, CHUNK)], sidx1, isem1)
        pltpu.make_async_copy(src_hbm.at[pl.ds(base, CHUNK)], sidx0,
                              isem0).wait()
        pltpu.async_copy(hs_hbm.at[sidx0], rows0, gsem0)
        pltpu.make_async_copy(src_hbm.at[pl.ds(base + CHUNK, CHUNK)], sidx1,
                              isem1).wait()
        pltpu.async_copy(hs_hbm.at[sidx1], rows1, gsem1)

        def chunk_step(j, sidx, rows, gsem, isem):
            pltpu.make_async_copy(hs_hbm.at[sidx], rows, gsem).wait()

            @pl.when(j + 2 < n_chunks)
            def _():
                pltpu.async_copy(
                    src_hbm.at[pl.ds(base + (j + 2) * CHUNK, CHUNK)],
                    sidx, isem)

            pltpu.sync_copy(rows, acc_sh.at[didx_all.at[j]], add=True)

            @pl.when(j + 2 < n_chunks)
            def _():
                pltpu.make_async_copy(
                    src_hbm.at[pl.ds(base + (j + 2) * CHUNK, CHUNK)],
                    sidx, isem).wait()
                pltpu.async_copy(hs_hbm.at[sidx], rows, gsem)

        def pair(t, carry):
            j0 = 2 * t
            chunk_step(j0, sidx0, rows0, gsem0, isem0)
            chunk_step(j0 + 1, sidx1, rows1, gsem1, isem1)
            return carry

        lax.fori_loop(0, n_chunks // 2, pair, 0)
        if n_chunks % 2:
            j = n_chunks - 1
            pltpu.make_async_copy(hs_hbm.at[sidx0], rows0, gsem0).wait()
            pltpu.sync_copy(rows0, acc_sh.at[didx_all.at[j]], add=True)
        plsc.subcore_barrier()
        pltpu.sync_copy(acc_sh.at[pl.ds(s * RPT_AGG, RPT_AGG)],
                        out_hbm.at[c, pl.ds(s * RPT_AGG, RPT_AGG)])

    return agg_kernel


_BLK = 2000


def _k1_body(x_ref, w_ref, dega_ref, degb_ref, hs_ref, dis_ref):
    deg = dega_ref[...] + degb_ref[...] + 1.0
    dis = lax.rsqrt(deg)
    h = jnp.dot(x_ref[...], w_ref[...], preferred_element_type=jnp.float32)
    hs_ref[...] = h * dis
    dis_ref[...] = dis


def _tc_k1(x, w1, dega, degb):
    grid = (N // _BLK,)
    return pl.pallas_call(
        _k1_body,
        grid=grid,
        in_specs=[
            pl.BlockSpec((_BLK, D), lambda i: (i, 0)),
            pl.BlockSpec((D, D), lambda i: (0, 0)),
            pl.BlockSpec((_BLK, 1), lambda i: (i, 0)),
            pl.BlockSpec((_BLK, 1), lambda i: (i, 0)),
        ],
        out_specs=[
            pl.BlockSpec((_BLK, D), lambda i: (i, 0)),
            pl.BlockSpec((_BLK, 1), lambda i: (i, 0)),
        ],
        out_shape=[
            jax.ShapeDtypeStruct((N, D), jnp.float32),
            jax.ShapeDtypeStruct((N, 1), jnp.float32),
        ],
    )(x, w1, dega, degb)


def _k2_body(a0_ref, a1_ref, hs_ref, dis_ref, b_ref, w_ref, out_ref):
    dis = dis_ref[...]
    t = dis * (a0_ref[...] + a1_ref[...] + hs_ref[...]) + b_ref[...]
    t = jnp.maximum(t, 0.0)
    h2 = jnp.dot(t, w_ref[...], preferred_element_type=jnp.float32)
    out_ref[...] = h2 * dis


def _tc_k2(a0, a1, hs, dis, b1, w2):
    grid = (N // _BLK,)
    return pl.pallas_call(
        _k2_body,
        grid=grid,
        in_specs=[
            pl.BlockSpec((_BLK, D), lambda i: (i, 0)),
            pl.BlockSpec((_BLK, D), lambda i: (i, 0)),
            pl.BlockSpec((_BLK, D), lambda i: (i, 0)),
            pl.BlockSpec((_BLK, 1), lambda i: (i, 0)),
            pl.BlockSpec((1, D), lambda i: (0, 0)),
            pl.BlockSpec((D, D), lambda i: (0, 0)),
        ],
        out_specs=pl.BlockSpec((_BLK, D), lambda i: (i, 0)),
        out_shape=jax.ShapeDtypeStruct((N, D), jnp.float32),
    )(a0, a1, hs, dis, b1, w2)


def _k3_body(a0_ref, a1_ref, hs_ref, dis_ref, b_ref, out_ref):
    out_ref[...] = (dis_ref[...] * (a0_ref[...] + a1_ref[...] + hs_ref[...])
                    + b_ref[...])


def _tc_k3(a0, a1, hs, dis, b2):
    grid = (N // _BLK,)
    return pl.pallas_call(
        _k3_body,
        grid=grid,
        in_specs=[
            pl.BlockSpec((_BLK, D), lambda i: (i, 0)),
            pl.BlockSpec((_BLK, D), lambda i: (i, 0)),
            pl.BlockSpec((_BLK, D), lambda i: (i, 0)),
            pl.BlockSpec((_BLK, 1), lambda i: (i, 0)),
            pl.BlockSpec((1, D), lambda i: (0, 0)),
        ],
        out_specs=pl.BlockSpec((_BLK, D), lambda i: (i, 0)),
        out_shape=jax.ShapeDtypeStruct((N, D), jnp.float32),
    )(a0, a1, hs, dis, b2)


@jax.jit
def kernel(x, edge_index, W1, b1, W2, b2):
    e = edge_index.shape[1]
    per_tile_chunks = -(-e // (NW * CHUNK))
    e_pad = per_tile_chunks * NW * CHUNK
    pad = e_pad - e
    n_chunks = e_pad // (NW * CHUNK)
    src = jnp.concatenate([edge_index[0], jnp.zeros((pad,), jnp.int32)])
    dst = jnp.concatenate([edge_index[1], jnp.full((pad,), N, jnp.int32)])
    dst = dst.reshape(NW, n_chunks, CHUNK)

    zeros1 = jnp.zeros((ROWS_PAD,), jnp.float32)
    zeros2 = jnp.zeros((ROWS_AGG, D), jnp.float32)

    deg_p = _make_deg_kernel(e_pad)(dst, zeros1)
    dega = deg_p[0, :N].reshape(N, 1)
    degb = deg_p[1, :N].reshape(N, 1)

    hs1, dis = _tc_k1(x, W1, dega, degb)

    agg_fn = _make_agg_kernel(e_pad)
    agg1 = agg_fn(hs1, src, dst, zeros2)
    hs2 = _tc_k2(agg1[0, :N], agg1[1, :N], hs1, dis,
                 b1.reshape(1, D), W2)

    agg2 = agg_fn(hs2, src, dst, zeros2)
    out = _tc_k3(agg2[0, :N], agg2[1, :N], hs2, dis, b2.reshape(1, D))
    return out


# speedup vs baseline: 2.7885x; 1.3552x over previous
import functools

import jax
import jax.numpy as jnp
from jax import lax
from jax.experimental import pallas as pl
from jax.experimental.pallas import tpu as pltpu
from jax.experimental.pallas import tpu_sc as plsc

N = 10000
D = 128
NC = 2
NS = 16
NW = NC * NS
CHUNK = 80
ROWS_PAD = 10240
RPT = ROWS_PAD // NS
ROWS_AGG = 10112
RPT_AGG = ROWS_AGG // NS


def _sc_mesh():
    return plsc.VectorSubcoreMesh(core_axis_name="c", subcore_axis_name="s")


_DEG_WIN = 4


def _make_deg_kernel(e_pad):
    ept = e_pad // NW
    n_chunks = ept // CHUNK

    @functools.partial(
        pl.kernel,
        out_type=jax.ShapeDtypeStruct((NC, ROWS_PAD), jnp.float32),
        mesh=_sc_mesh(),
        scratch_types=[
            pltpu.VMEM((n_chunks, CHUNK), jnp.int32),
            pltpu.VMEM((CHUNK,), jnp.float32),
            pltpu.VMEM_SHARED((ROWS_PAD,), jnp.float32),
            pltpu.SemaphoreType.DMA,
        ],
    )
    def deg_kernel(dst_hbm, zeros_hbm, out_hbm, didx_all, ones_v, acc_sh, sem):
        c = lax.axis_index("c")
        s = lax.axis_index("s")
        wid = s * NC + c
        pltpu.sync_copy(zeros_hbm.at[pl.ds(s * RPT, RPT)],
                        acc_sh.at[pl.ds(s * RPT, RPT)])
        for i in range(CHUNK // 16):
            ones_v[pl.ds(i * 16, 16)] = jnp.ones((16,), jnp.float32)
        pltpu.sync_copy(dst_hbm.at[wid], didx_all)
        plsc.subcore_barrier()

        def body(j, carry):
            pltpu.async_copy(ones_v, acc_sh.at[didx_all.at[j]], sem, add=True)

            @pl.when(j >= _DEG_WIN)
            def _():
                pltpu.make_async_copy(
                    ones_v, acc_sh.at[didx_all.at[0]], sem).wait()

            return carry

        lax.fori_loop(0, n_chunks, body, 0)
        for _ in range(min(_DEG_WIN, n_chunks)):
            pltpu.make_async_copy(ones_v, acc_sh.at[didx_all.at[0]], sem).wait()
        plsc.subcore_barrier()
        pltpu.sync_copy(acc_sh.at[pl.ds(s * RPT, RPT)],
                        out_hbm.at[c, pl.ds(s * RPT, RPT)])

    return deg_kernel


def _make_agg_kernel(e_pad):
    ept = e_pad // NW
    n_chunks = ept // CHUNK

    @functools.partial(
        pl.kernel,
        out_type=jax.ShapeDtypeStruct((NC, ROWS_AGG, D), jnp.float32),
        mesh=_sc_mesh(),
        scratch_types=[
            pltpu.VMEM((n_chunks, CHUNK), jnp.int32),
            pltpu.VMEM((CHUNK,), jnp.int32),
            pltpu.VMEM((CHUNK,), jnp.int32),
            pltpu.VMEM((CHUNK,), jnp.int32),
            pltpu.VMEM((CHUNK, D), jnp.float32),
            pltpu.VMEM((CHUNK, D), jnp.float32),
            pltpu.VMEM((CHUNK, D), jnp.float32),
            pltpu.VMEM_SHARED((ROWS_AGG, D), jnp.float32),
            pltpu.SemaphoreType.DMA,
            pltpu.SemaphoreType.DMA,
            pltpu.SemaphoreType.DMA,
            pltpu.SemaphoreType.DMA,
            pltpu.SemaphoreType.DMA,
            pltpu.SemaphoreType.DMA,
        ],
    )
    def agg_kernel(hs_hbm, src_hbm, dst_hbm, zeros_hbm, out_hbm,
                   didx_all, sidx0, sidx1, sidx2, rows0, rows1, rows2, acc_sh,
                   gsem0, gsem1, gsem2, isem0, isem1, isem2):
        c = lax.axis_index("c")
        s = lax.axis_index("s")
        wid = s * NC + c
        base = wid * ept
        pltpu.sync_copy(zeros_hbm.at[pl.ds(s * RPT_AGG, RPT_AGG)],
                        acc_sh.at[pl.ds(s * RPT_AGG, RPT_AGG)])
        pltpu.sync_copy(dst_hbm.at[wid], didx_all)
        plsc.subcore_barrier()

        sidx = (sidx0, sidx1, sidx2)
        rows = (rows0, rows1, rows2)
        gsem = (gsem0, gsem1, gsem2)
        isem = (isem0, isem1, isem2)
        for k in range(3):
            pltpu.async_copy(src_hbm.at[pl.ds(base + k * CHUNK, CHUNK)],
                             sidx[k], isem[k])
        for k in range(3):
            pltpu.make_async_copy(src_hbm.at[pl.ds(base, CHUNK)],
                                  sidx[k], isem[k]).wait()
            pltpu.async_copy(hs_hbm.at[sidx[k]], rows[k], gsem[k])

        def chunk_step(j, k):
            pltpu.make_async_copy(hs_hbm.at[sidx[k]], rows[k],
                                  gsem[k]).wait()

            @pl.when(j + 3 < n_chunks)
            def _():
                pltpu.async_copy(
                    src_hbm.at[pl.ds(base + (j + 3) * CHUNK, CHUNK)],
                    sidx[k], isem[k])

            pltpu.sync_copy(rows[k], acc_sh.at[didx_all.at[j]], add=True)

            @pl.when(j + 3 < n_chunks)
            def _():
                pltpu.make_async_copy(src_hbm.at[pl.ds(base, CHUNK)],
                                      sidx[k], isem[k]).wait()
                pltpu.async_copy(hs_hbm.at[sidx[k]], rows[k], gsem[k])

        def triple(t, carry):
            j0 = 3 * t
            chunk_step(j0, 0)
            chunk_step(j0 + 1, 1)
            chunk_step(j0 + 2, 2)
            return carry

        lax.fori_loop(0, n_chunks // 3, triple, 0)
        plsc.subcore_barrier()
        pltpu.sync_copy(acc_sh.at[pl.ds(s * RPT_AGG, RPT_AGG)],
                        out_hbm.at[c, pl.ds(s * RPT_AGG, RPT_AGG)])

    return agg_kernel


_BLK = 2000


def _k1_body(x_ref, w_ref, dega_ref, degb_ref, hs_ref, dis_ref):
    deg = dega_ref[...] + degb_ref[...] + 1.0
    dis = lax.rsqrt(deg)
    h = jnp.dot(x_ref[...], w_ref[...], preferred_element_type=jnp.float32)
    hs_ref[...] = h * dis
    dis_ref[...] = dis


def _tc_k1(x, w1, dega, degb):
    grid = (N // _BLK,)
    return pl.pallas_call(
        _k1_body,
        grid=grid,
        in_specs=[
            pl.BlockSpec((_BLK, D), lambda i: (i, 0)),
            pl.BlockSpec((D, D), lambda i: (0, 0)),
            pl.BlockSpec((_BLK, 1), lambda i: (i, 0)),
            pl.BlockSpec((_BLK, 1), lambda i: (i, 0)),
        ],
        out_specs=[
            pl.BlockSpec((_BLK, D), lambda i: (i, 0)),
            pl.BlockSpec((_BLK, 1), lambda i: (i, 0)),
        ],
        out_shape=[
            jax.ShapeDtypeStruct((N, D), jnp.float32),
            jax.ShapeDtypeStruct((N, 1), jnp.float32),
        ],
    )(x, w1, dega, degb)


def _k2_body(a0_ref, a1_ref, hs_ref, dis_ref, b_ref, w_ref, out_ref):
    dis = dis_ref[...]
    t = dis * (a0_ref[...] + a1_ref[...] + hs_ref[...]) + b_ref[...]
    t = jnp.maximum(t, 0.0)
    h2 = jnp.dot(t, w_ref[...], preferred_element_type=jnp.float32)
    out_ref[...] = h2 * dis


def _tc_k2(a0, a1, hs, dis, b1, w2):
    grid = (N // _BLK,)
    return pl.pallas_call(
        _k2_body,
        grid=grid,
        in_specs=[
            pl.BlockSpec((_BLK, D), lambda i: (i, 0)),
            pl.BlockSpec((_BLK, D), lambda i: (i, 0)),
            pl.BlockSpec((_BLK, D), lambda i: (i, 0)),
            pl.BlockSpec((_BLK, 1), lambda i: (i, 0)),
            pl.BlockSpec((1, D), lambda i: (0, 0)),
            pl.BlockSpec((D, D), lambda i: (0, 0)),
        ],
        out_specs=pl.BlockSpec((_BLK, D), lambda i: (i, 0)),
        out_shape=jax.ShapeDtypeStruct((N, D), jnp.float32),
    )(a0, a1, hs, dis, b1, w2)


def _k3_body(a0_ref, a1_ref, hs_ref, dis_ref, b_ref, out_ref):
    out_ref[...] = (dis_ref[...] * (a0_ref[...] + a1_ref[...] + hs_ref[...])
                    + b_ref[...])


def _tc_k3(a0, a1, hs, dis, b2):
    grid = (N // _BLK,)
    return pl.pallas_call(
        _k3_body,
        grid=grid,
        in_specs=[
            pl.BlockSpec((_BLK, D), lambda i: (i, 0)),
            pl.BlockSpec((_BLK, D), lambda i: (i, 0)),
            pl.BlockSpec((_BLK, D), lambda i: (i, 0)),
            pl.BlockSpec((_BLK, 1), lambda i: (i, 0)),
            pl.BlockSpec((1, D), lambda i: (0, 0)),
        ],
        out_specs=pl.BlockSpec((_BLK, D), lambda i: (i, 0)),
        out_shape=jax.ShapeDtypeStruct((N, D), jnp.float32),
    )(a0, a1, hs, dis, b2)


@jax.jit
def kernel(x, edge_index, W1, b1, W2, b2):
    e = edge_index.shape[1]
    per_tile_chunks = -(-e // (NW * CHUNK))
    per_tile_chunks = -(-per_tile_chunks // 3) * 3
    e_pad = per_tile_chunks * NW * CHUNK
    pad = e_pad - e
    n_chunks = e_pad // (NW * CHUNK)
    src = jnp.concatenate([edge_index[0], jnp.zeros((pad,), jnp.int32)])
    dst = jnp.concatenate([edge_index[1], jnp.full((pad,), N, jnp.int32)])
    dst = dst.reshape(NW, n_chunks, CHUNK)

    zeros1 = jnp.zeros((ROWS_PAD,), jnp.float32)
    zeros2 = jnp.zeros((ROWS_AGG, D), jnp.float32)

    deg_p = _make_deg_kernel(e_pad)(dst, zeros1)
    dega = deg_p[0, :N].reshape(N, 1)
    degb = deg_p[1, :N].reshape(N, 1)

    hs1, dis = _tc_k1(x, W1, dega, degb)

    agg_fn = _make_agg_kernel(e_pad)
    agg1 = agg_fn(hs1, src, dst, zeros2)
    hs2 = _tc_k2(agg1[0, :N], agg1[1, :N], hs1, dis,
                 b1.reshape(1, D), W2)

    agg2 = agg_fn(hs2, src, dst, zeros2)
    out = _tc_k3(agg2[0, :N], agg2[1, :N], hs2, dis, b2.reshape(1, D))
    return out


# speedup vs baseline: 3.6661x; 1.3147x over previous
import functools

import jax
import jax.numpy as jnp
from jax import lax
from jax.experimental import pallas as pl
from jax.experimental.pallas import tpu as pltpu
from jax.experimental.pallas import tpu_sc as plsc

N = 10000
D = 128
NC = 2
NS = 16
NW = NC * NS
CHUNK = 88
CHUNK_D = 80
ROWS_PAD = 10240
RPT = ROWS_PAD // NS
ROWS_AGG = 10112
RPT_AGG = ROWS_AGG // NS


def _sc_mesh():
    return plsc.VectorSubcoreMesh(core_axis_name="c", subcore_axis_name="s")


_DEG_WIN = 4


def _make_deg_kernel(e_pad):
    ept = e_pad // NW
    n_chunks = ept // CHUNK_D

    @functools.partial(
        pl.kernel,
        out_type=jax.ShapeDtypeStruct((NC, ROWS_PAD), jnp.float32),
        mesh=_sc_mesh(),
        scratch_types=[
            pltpu.VMEM((n_chunks, CHUNK_D), jnp.int32),
            pltpu.VMEM((CHUNK_D,), jnp.float32),
            pltpu.VMEM_SHARED((ROWS_PAD,), jnp.float32),
            pltpu.SemaphoreType.DMA,
        ],
    )
    def deg_kernel(dst_hbm, zeros_hbm, out_hbm, didx_all, ones_v, acc_sh, sem):
        c = lax.axis_index("c")
        s = lax.axis_index("s")
        wid = s * NC + c
        pltpu.sync_copy(zeros_hbm.at[pl.ds(s * RPT, RPT)],
                        acc_sh.at[pl.ds(s * RPT, RPT)])
        for i in range(CHUNK_D // 16):
            ones_v[pl.ds(i * 16, 16)] = jnp.ones((16,), jnp.float32)
        pltpu.sync_copy(dst_hbm.at[wid], didx_all)
        plsc.subcore_barrier()

        def body(j, carry):
            pltpu.async_copy(ones_v, acc_sh.at[didx_all.at[j]], sem, add=True)

            @pl.when(j >= _DEG_WIN)
            def _():
                pltpu.make_async_copy(
                    ones_v, acc_sh.at[didx_all.at[0]], sem).wait()

            return carry

        lax.fori_loop(0, n_chunks, body, 0)
        for _ in range(min(_DEG_WIN, n_chunks)):
            pltpu.make_async_copy(ones_v, acc_sh.at[didx_all.at[0]], sem).wait()
        plsc.subcore_barrier()
        pltpu.sync_copy(acc_sh.at[pl.ds(s * RPT, RPT)],
                        out_hbm.at[c, pl.ds(s * RPT, RPT)])

    return deg_kernel


def _make_agg_kernel(e_pad):
    ept = e_pad // NW
    n_chunks = ept // CHUNK

    @functools.partial(
        pl.kernel,
        out_type=jax.ShapeDtypeStruct((NC, ROWS_AGG, D), jnp.float32),
        mesh=_sc_mesh(),
        scratch_types=[
            pltpu.VMEM((n_chunks, CHUNK), jnp.int32),
            pltpu.VMEM((CHUNK,), jnp.int32),
            pltpu.VMEM((CHUNK,), jnp.int32),
            pltpu.VMEM((CHUNK,), jnp.int32),
            pltpu.VMEM((CHUNK, D), jnp.float32),
            pltpu.VMEM((CHUNK, D), jnp.float32),
            pltpu.VMEM((CHUNK, D), jnp.float32),
            pltpu.VMEM_SHARED((ROWS_AGG, D), jnp.float32),
            pltpu.SemaphoreType.DMA,
            pltpu.SemaphoreType.DMA,
            pltpu.SemaphoreType.DMA,
            pltpu.SemaphoreType.DMA,
            pltpu.SemaphoreType.DMA,
            pltpu.SemaphoreType.DMA,
        ],
    )
    def agg_kernel(hs_hbm, src_hbm, dst_hbm, zeros_hbm, out_hbm,
                   didx_all, sidx0, sidx1, sidx2, rows0, rows1, rows2, acc_sh,
                   gsem0, gsem1, gsem2, isem0, isem1, isem2):
        c = lax.axis_index("c")
        s = lax.axis_index("s")
        wid = s * NC + c
        base = wid * ept
        pltpu.sync_copy(zeros_hbm.at[pl.ds(s * RPT_AGG, RPT_AGG)],
                        acc_sh.at[pl.ds(s * RPT_AGG, RPT_AGG)])
        pltpu.sync_copy(dst_hbm.at[wid], didx_all)
        plsc.subcore_barrier()

        sidx = (sidx0, sidx1, sidx2)
        rows = (rows0, rows1, rows2)
        gsem = (gsem0, gsem1, gsem2)
        isem = (isem0, isem1, isem2)
        for k in range(3):
            pltpu.async_copy(src_hbm.at[pl.ds(base + k * CHUNK, CHUNK)],
                             sidx[k], isem[k])
        for k in range(3):
            pltpu.make_async_copy(src_hbm.at[pl.ds(base, CHUNK)],
                                  sidx[k], isem[k]).wait()
            pltpu.async_copy(hs_hbm.at[sidx[k]], rows[k], gsem[k])

        def chunk_step(j, k):
            pltpu.make_async_copy(hs_hbm.at[sidx[k]], rows[k],
                                  gsem[k]).wait()

            @pl.when(j + 3 < n_chunks)
            def _():
                pltpu.async_copy(
                    src_hbm.at[pl.ds(base + (j + 3) * CHUNK, CHUNK)],
                    sidx[k], isem[k])

            pltpu.sync_copy(rows[k], acc_sh.at[didx_all.at[j]], add=True)

            @pl.when(j + 3 < n_chunks)
            def _():
                pltpu.make_async_copy(src_hbm.at[pl.ds(base, CHUNK)],
                                      sidx[k], isem[k]).wait()
                pltpu.async_copy(hs_hbm.at[sidx[k]], rows[k], gsem[k])

        def triple(t, carry):
            j0 = 3 * t
            chunk_step(j0, 0)
            chunk_step(j0 + 1, 1)
            chunk_step(j0 + 2, 2)
            return carry

        lax.fori_loop(0, n_chunks // 3, triple, 0)
        plsc.subcore_barrier()
        pltpu.sync_copy(acc_sh.at[pl.ds(s * RPT_AGG, RPT_AGG)],
                        out_hbm.at[c, pl.ds(s * RPT_AGG, RPT_AGG)])

    return agg_kernel


_BLK = 2000


def _k1_body(x_ref, w_ref, dega_ref, degb_ref, hs_ref, dis_ref):
    deg = dega_ref[...] + degb_ref[...] + 1.0
    dis = lax.rsqrt(deg)
    h = jnp.dot(x_ref[...], w_ref[...], preferred_element_type=jnp.float32)
    hs_ref[...] = h * dis
    dis_ref[...] = dis


def _tc_k1(x, w1, dega, degb):
    grid = (N // _BLK,)
    return pl.pallas_call(
        _k1_body,
        grid=grid,
        in_specs=[
            pl.BlockSpec((_BLK, D), lambda i: (i, 0)),
            pl.BlockSpec((D, D), lambda i: (0, 0)),
            pl.BlockSpec((_BLK, 1), lambda i: (i, 0)),
            pl.BlockSpec((_BLK, 1), lambda i: (i, 0)),
        ],
        out_specs=[
            pl.BlockSpec((_BLK, D), lambda i: (i, 0)),
            pl.BlockSpec((_BLK, 1), lambda i: (i, 0)),
        ],
        out_shape=[
            jax.ShapeDtypeStruct((N, D), jnp.float32),
            jax.ShapeDtypeStruct((N, 1), jnp.float32),
        ],
    )(x, w1, dega, degb)


def _k2_body(a0_ref, a1_ref, hs_ref, dis_ref, b_ref, w_ref, out_ref):
    dis = dis_ref[...]
    t = dis * (a0_ref[...] + a1_ref[...] + hs_ref[...]) + b_ref[...]
    t = jnp.maximum(t, 0.0)
    h2 = jnp.dot(t, w_ref[...], preferred_element_type=jnp.float32)
    out_ref[...] = h2 * dis


def _tc_k2(a0, a1, hs, dis, b1, w2):
    grid = (N // _BLK,)
    return pl.pallas_call(
        _k2_body,
        grid=grid,
        in_specs=[
            pl.BlockSpec((_BLK, D), lambda i: (i, 0)),
            pl.BlockSpec((_BLK, D), lambda i: (i, 0)),
            pl.BlockSpec((_BLK, D), lambda i: (i, 0)),
            pl.BlockSpec((_BLK, 1), lambda i: (i, 0)),
            pl.BlockSpec((1, D), lambda i: (0, 0)),
            pl.BlockSpec((D, D), lambda i: (0, 0)),
        ],
        out_specs=pl.BlockSpec((_BLK, D), lambda i: (i, 0)),
        out_shape=jax.ShapeDtypeStruct((N, D), jnp.float32),
    )(a0, a1, hs, dis, b1, w2)


def _k3_body(a0_ref, a1_ref, hs_ref, dis_ref, b_ref, out_ref):
    out_ref[...] = (dis_ref[...] * (a0_ref[...] + a1_ref[...] + hs_ref[...])
                    + b_ref[...])


def _tc_k3(a0, a1, hs, dis, b2):
    grid = (N // _BLK,)
    return pl.pallas_call(
        _k3_body,
        grid=grid,
        in_specs=[
            pl.BlockSpec((_BLK, D), lambda i: (i, 0)),
            pl.BlockSpec((_BLK, D), lambda i: (i, 0)),
            pl.BlockSpec((_BLK, D), lambda i: (i, 0)),
            pl.BlockSpec((_BLK, 1), lambda i: (i, 0)),
            pl.BlockSpec((1, D), lambda i: (0, 0)),
        ],
        out_specs=pl.BlockSpec((_BLK, D), lambda i: (i, 0)),
        out_shape=jax.ShapeDtypeStruct((N, D), jnp.float32),
    )(a0, a1, hs, dis, b2)


@jax.jit
def kernel(x, edge_index, W1, b1, W2, b2):
    e = edge_index.shape[1]
    per_tile_chunks = -(-e // (NW * CHUNK))
    per_tile_chunks = -(-per_tile_chunks // 3) * 3
    e_pad = per_tile_chunks * NW * CHUNK
    pad = e_pad - e
    n_chunks = e_pad // (NW * CHUNK)
    src = jnp.concatenate([edge_index[0], jnp.zeros((pad,), jnp.int32)])
    dst = jnp.concatenate([edge_index[1], jnp.full((pad,), N, jnp.int32)])
    dst = dst.reshape(NW, n_chunks, CHUNK)
    ed_pad = -(-e // (NW * CHUNK_D)) * NW * CHUNK_D
    dstd = jnp.concatenate(
        [edge_index[1], jnp.full((ed_pad - e,), N, jnp.int32)])
    dstd = dstd.reshape(NW, ed_pad // (NW * CHUNK_D), CHUNK_D)

    zeros1 = jnp.zeros((ROWS_PAD,), jnp.float32)
    zeros2 = jnp.zeros((ROWS_AGG, D), jnp.float32)

    deg_p = _make_deg_kernel(ed_pad)(dstd, zeros1)
    dega = deg_p[0, :N].reshape(N, 1)
    degb = deg_p[1, :N].reshape(N, 1)

    hs1, dis = _tc_k1(x, W1, dega, degb)

    agg_fn = _make_agg_kernel(e_pad)
    agg1 = agg_fn(hs1, src, dst, zeros2)
    hs2 = _tc_k2(agg1[0, :N], agg1[1, :N], hs1, dis,
                 b1.reshape(1, D), W2)

    agg2 = agg_fn(hs2, src, dst, zeros2)
    out = _tc_k3(agg2[0, :N], agg2[1, :N], hs2, dis, b2.reshape(1, D))
    return out
